# baseline (device time: 24626 ns/iter reference)
import jax
import jax.numpy as jnp
from jax import lax
from jax.experimental import pallas as pl
from jax.experimental.pallas import tpu as pltpu

N_DEV = 16


def kernel(x):
    _, m, n = x.shape
    c_rows = m // N_DEV

    def body(x_ref, out_ref, acc_ref, rbuf, send_sems, recv_sems):
        i = lax.axis_index("i")

        acc_ref[...] = x_ref[0].astype(jnp.bfloat16)

        barrier_sem = pltpu.get_barrier_semaphore()
        for d in range(1, N_DEV):
            pl.semaphore_signal(
                barrier_sem, inc=1,
                device_id=((i + d) % N_DEV,),
                device_id_type=pl.DeviceIdType.MESH,
            )
        pl.semaphore_wait(barrier_sem, N_DEV - 1)

        all_sends = []

        def dummy_recv(buf, slot):
            return pltpu.make_async_remote_copy(
                src_ref=buf, dst_ref=buf,
                send_sem=send_sems.at[slot], recv_sem=recv_sems.at[slot],
                device_id=(i,), device_id_type=pl.DeviceIdType.MESH,
            )

        far_first = (8, 7, 9, 6, 10, 5, 11, 4, 12, 3, 13, 2, 14, 1, 15)

        for d in far_first:
            pt = (i + d) % N_DEV
            r = pltpu.make_async_remote_copy(
                src_ref=acc_ref.at[pl.ds(pt * c_rows, c_rows)],
                dst_ref=rbuf.at[N_DEV - 1 - d],
                send_sem=send_sems.at[d - 1],
                recv_sem=recv_sems.at[N_DEV - 1 - d],
                device_id=(pt,),
                device_id_type=pl.DeviceIdType.MESH,
            )
            r.start()
            all_sends.append(r)

        my_off = i * c_rows
        for slot in (14, 13, 12, 11, 10):
            dummy_recv(rbuf.at[slot], slot).wait_recv()
        acc_ref[pl.ds(my_off, c_rows)] = (
            acc_ref[pl.ds(my_off, c_rows)]
            + rbuf[14] + rbuf[13] + rbuf[12] + rbuf[11] + rbuf[10]
        )
        for slot in (9, 8, 7, 6, 5):
            dummy_recv(rbuf.at[slot], slot).wait_recv()
        acc_ref[pl.ds(my_off, c_rows)] = (
            acc_ref[pl.ds(my_off, c_rows)]
            + rbuf[9] + rbuf[8] + rbuf[7] + rbuf[6] + rbuf[5]
        )
        for slot in (4, 3, 2, 1, 0):
            dummy_recv(rbuf.at[slot], slot).wait_recv()
        acc_ref[pl.ds(my_off, c_rows)] = (
            acc_ref[pl.ds(my_off, c_rows)]
            + rbuf[4] + rbuf[3] + rbuf[2] + rbuf[1] + rbuf[0]
        )

        for d in far_first:
            pt = (i + d) % N_DEV
            r = pltpu.make_async_remote_copy(
                src_ref=acc_ref.at[pl.ds(my_off, c_rows)],
                dst_ref=acc_ref.at[pl.ds(my_off, c_rows)],
                send_sem=send_sems.at[N_DEV - 1 + d - 1],
                recv_sem=recv_sems.at[2 * (N_DEV - 1) - d],
                device_id=(pt,),
                device_id_type=pl.DeviceIdType.MESH,
            )
            r.start()
            all_sends.append(r)

        out_ref[pl.ds(my_off, c_rows)] = acc_ref[pl.ds(my_off, c_rows)]
        for s in range(N_DEV - 2, -1, -1):
            dummy_recv(
                acc_ref.at[pl.ds(my_off, c_rows)], N_DEV - 1 + s
            ).wait_recv()
            off = ((i - (N_DEV - 1 - s)) % N_DEV) * c_rows
            out_ref[pl.ds(off, c_rows)] = acc_ref[pl.ds(off, c_rows)]

        for r in all_sends:
            r.wait_send()

    return pl.pallas_call(
        body,
        out_shape=jax.ShapeDtypeStruct((m, n), jnp.bfloat16),
        in_specs=[pl.BlockSpec(memory_space=pltpu.VMEM)],
        out_specs=pl.BlockSpec(memory_space=pltpu.VMEM),
        scratch_shapes=[
            pltpu.VMEM((m, n), jnp.bfloat16),
            pltpu.VMEM((N_DEV - 1, c_rows, n), jnp.bfloat16),
            pltpu.SemaphoreType.DMA((2 * (N_DEV - 1),)),
            pltpu.SemaphoreType.DMA((2 * (N_DEV - 1),)),
        ],
        compiler_params=pltpu.CompilerParams(collective_id=0),
    )(x)


# device time: 21493 ns/iter; 1.1458x vs baseline; 1.1458x over previous
import jax
import jax.numpy as jnp
from jax import lax
from jax.experimental import pallas as pl
from jax.experimental.pallas import tpu as pltpu

N_DEV = 16


def kernel(x):
    _, m, n = x.shape
    c_rows = m // N_DEV

    def body(x_ref, out_ref, acc_ref, rbuf, send_sems, recv_sems):
        i = lax.axis_index("i")

        acc_ref[...] = x_ref[0].astype(jnp.bfloat16)

        barrier_sem = pltpu.get_barrier_semaphore()
        for d in range(1, N_DEV):
            pl.semaphore_signal(
                barrier_sem, inc=1,
                device_id=((i + d) % N_DEV,),
                device_id_type=pl.DeviceIdType.MESH,
            )
        pl.semaphore_wait(barrier_sem, N_DEV - 1)

        all_sends = []

        def dummy_recv(buf, slot):
            return pltpu.make_async_remote_copy(
                src_ref=buf, dst_ref=buf,
                send_sem=send_sems.at[slot], recv_sem=recv_sems.at[slot],
                device_id=(i,), device_id_type=pl.DeviceIdType.MESH,
            )

        for d in range(1, N_DEV):
            pt = (i + d) % N_DEV
            r = pltpu.make_async_remote_copy(
                src_ref=acc_ref.at[pl.ds(pt * c_rows, c_rows)],
                dst_ref=rbuf.at[N_DEV - 1 - d],
                send_sem=send_sems.at[d - 1],
                recv_sem=recv_sems.at[N_DEV - 1 - d],
                device_id=(pt,),
                device_id_type=pl.DeviceIdType.MESH,
            )
            r.start()
            all_sends.append(r)

        my_off = i * c_rows
        for slot in (14, 13, 12, 11, 10):
            dummy_recv(rbuf.at[slot], slot).wait_recv()
        acc_ref[pl.ds(my_off, c_rows)] = (
            acc_ref[pl.ds(my_off, c_rows)]
            + rbuf[14] + rbuf[13] + rbuf[12] + rbuf[11] + rbuf[10]
        )
        for slot in (9, 8, 7, 6, 5):
            dummy_recv(rbuf.at[slot], slot).wait_recv()
        acc_ref[pl.ds(my_off, c_rows)] = (
            acc_ref[pl.ds(my_off, c_rows)]
            + rbuf[9] + rbuf[8] + rbuf[7] + rbuf[6] + rbuf[5]
        )
        for slot in (4, 3, 2, 1, 0):
            dummy_recv(rbuf.at[slot], slot).wait_recv()
        acc_ref[pl.ds(my_off, c_rows)] = (
            acc_ref[pl.ds(my_off, c_rows)]
            + rbuf[4] + rbuf[3] + rbuf[2] + rbuf[1] + rbuf[0]
        )

        for d in range(1, N_DEV):
            pt = (i + d) % N_DEV
            r = pltpu.make_async_remote_copy(
                src_ref=acc_ref.at[pl.ds(my_off, c_rows)],
                dst_ref=acc_ref.at[pl.ds(my_off, c_rows)],
                send_sem=send_sems.at[N_DEV - 1 + d - 1],
                recv_sem=recv_sems.at[2 * (N_DEV - 1) - d],
                device_id=(pt,),
                device_id_type=pl.DeviceIdType.MESH,
            )
            r.start()
            all_sends.append(r)

        out_ref[pl.ds(my_off, c_rows)] = acc_ref[pl.ds(my_off, c_rows)]
        for s in range(N_DEV - 2, -1, -1):
            dummy_recv(
                acc_ref.at[pl.ds(my_off, c_rows)], N_DEV - 1 + s
            ).wait_recv()
            off = ((i - (N_DEV - 1 - s)) % N_DEV) * c_rows
            out_ref[pl.ds(off, c_rows)] = acc_ref[pl.ds(off, c_rows)]

        for r in all_sends:
            r.wait_send()

    return pl.pallas_call(
        body,
        out_shape=jax.ShapeDtypeStruct((m, n), jnp.bfloat16),
        in_specs=[pl.BlockSpec(memory_space=pltpu.VMEM)],
        out_specs=pl.BlockSpec(memory_space=pltpu.VMEM),
        scratch_shapes=[
            pltpu.VMEM((m, n), jnp.bfloat16),
            pltpu.VMEM((N_DEV - 1, c_rows, n), jnp.bfloat16),
            pltpu.SemaphoreType.DMA((2 * (N_DEV - 1),)),
            pltpu.SemaphoreType.DMA((2 * (N_DEV - 1),)),
        ],
        compiler_params=pltpu.CompilerParams(collective_id=0),
    )(x)
